# baseline (speedup 1.0000x reference)
import functools

import jax
import jax.numpy as jnp
from jax.experimental import pallas as pl
from jax.experimental.pallas import tpu as pltpu


@functools.lru_cache(maxsize=None)
def _build_tc_kernel(bs, h, w, F):
    F2 = 2 * F

    NHBM = 2

    def body(row_ref, col_ref, out_ref, img_vmem, sem, sem2):
        col = col_ref[pl.ds(0, w), :]
        row = row_ref[pl.ds(0, h), :]
        img_vmem[:, :, 0:F] = jnp.broadcast_to(col[None, :, :], (h, w, F))
        img_vmem[:, :, F:F2] = jnp.broadcast_to(row[:, None, :], (h, w, F))
        for b in range(NHBM):
            pltpu.make_async_copy(img_vmem, out_ref.at[b], sem2).start()
        for b in range(NHBM, bs - NHBM):
            pltpu.make_async_copy(img_vmem, out_ref.at[b], sem).start()
        for b in range(NHBM):
            pltpu.make_async_copy(img_vmem, out_ref.at[b], sem2).wait()
        for i in range(NHBM):
            pltpu.make_async_copy(
                out_ref.at[i], out_ref.at[bs - NHBM + i], sem).start()
        for b in range(NHBM, bs):
            pltpu.make_async_copy(img_vmem, out_ref.at[b], sem).wait()

    return pl.pallas_call(
        body,
        out_shape=jax.ShapeDtypeStruct((bs, h, w, F2), jnp.float32),
        in_specs=[
            pl.BlockSpec(memory_space=pltpu.VMEM),
            pl.BlockSpec(memory_space=pltpu.VMEM),
        ],
        out_specs=pl.BlockSpec(memory_space=pl.ANY),
        scratch_shapes=[
            pltpu.VMEM((h, w, F2), jnp.float32),
            pltpu.SemaphoreType.DMA,
            pltpu.SemaphoreType.DMA,
        ],
    )


def kernel(mask, row_embed, col_embed):
    bs, h, w = mask.shape
    F = row_embed.shape[1]
    tc_kernel = _build_tc_kernel(bs, h, w, F)
    out_bhwf = tc_kernel(row_embed, col_embed)
    return jnp.transpose(out_bhwf, (0, 3, 1, 2))


# speedup vs baseline: 5.5805x; 5.5805x over previous
import functools

import jax
import jax.numpy as jnp
from jax.experimental import pallas as pl
from jax.experimental.pallas import tpu as pltpu


@functools.lru_cache(maxsize=None)
def _build_tc_kernel(bs, h, w, F):
    F2 = 2 * F

    def body(row_ref, col_ref, out_ref, img_vmem, sem):
        col = col_ref[pl.ds(0, w), :]
        row = row_ref[pl.ds(0, h), :]
        img_vmem[:, :, 0:F] = jnp.broadcast_to(col[None, :, :], (h, w, F))
        img_vmem[:, :, F:F2] = jnp.broadcast_to(row[:, None, :], (h, w, F))
        for b in range(bs):
            pltpu.make_async_copy(img_vmem, out_ref.at[b], sem).start()
        for b in range(bs):
            pltpu.make_async_copy(img_vmem, out_ref.at[b], sem).wait()

    return pl.pallas_call(
        body,
        out_shape=jax.ShapeDtypeStruct((bs, h, w, F2), jnp.float32),
        in_specs=[
            pl.BlockSpec(memory_space=pltpu.VMEM),
            pl.BlockSpec(memory_space=pltpu.VMEM),
        ],
        out_specs=pl.BlockSpec(memory_space=pl.ANY),
        scratch_shapes=[
            pltpu.VMEM((h, w, F2), jnp.float32),
            pltpu.SemaphoreType.DMA,
        ],
    )


def kernel(mask, row_embed, col_embed):
    bs, h, w = mask.shape
    F = row_embed.shape[1]
    tc_kernel = _build_tc_kernel(bs, h, w, F)
    out_bhwf = tc_kernel(row_embed, col_embed)
    return jnp.transpose(out_bhwf, (0, 3, 1, 2))
